# Initial kernel scaffold; baseline (speedup 1.0000x reference)
import jax
import jax.numpy as jnp
from jax.experimental import pallas as pl


def kernel(x, edge_attr, W1a, b1a, g1, be1, W1b, b1b, W2a, b2a, g2, be2, W2b, b2b, Wl1, bl1, Wl2, bl2, edge_index):
    raise NotImplementedError("write your pallas kernel here")


# speedup vs baseline: 6.4074x; 6.4074x over previous
import functools

import jax
import jax.numpy as jnp
import numpy as np
from jax import lax
from jax.experimental import pallas as pl
from jax.experimental.pallas import tpu as pltpu
from jax.experimental.pallas import tpu_sc as plsc

N = 50000
E = 800000
NC = 2
NS = 16
EP = 819200
PAD = EP - E
NROWS = EP // 128
ACC_PAD = 3128 * 16
TPT = 3128
LAST = N - 15 * TPT

_mesh = plsc.VectorSubcoreMesh(core_axis_name="c", subcore_axis_name="s",
                               num_cores=NC, num_subcores=NS)


def _s1_body(x_hbm, src_hbm, dst_hbm, p_hbm,
             srcb, vals, dstb, zbuf, acc, sem_st, sem_g, sem_sc):
    c = lax.axis_index("c")
    t = lax.axis_index("s")
    base = t * TPT

    def _z(i, _):
        zbuf[pl.ds(i * 16, 16)] = jnp.zeros((16,), jnp.float32)
        return _
    lax.fori_loop(0, TPT // 16 + 1, _z, None)
    pltpu.sync_copy(zbuf.at[pl.ds(0, TPT)], acc.at[pl.ds(base, TPT)])
    plsc.subcore_barrier()

    row0 = c * 3200 + t * 200

    def _chunk(ch, _):
        erow = row0 + ch * 8
        cp1 = pltpu.async_copy(src_hbm.at[pl.ds(erow * 128, 1024)], srcb, sem_st)
        cp2 = pltpu.async_copy(dst_hbm.at[pl.ds(erow, 8)], dstb, sem_st)
        cp1.wait()
        cp2.wait()
        gs = [pltpu.async_copy(x_hbm.at[srcb.at[pl.ds(b * 128, 128)]],
                               vals.at[pl.ds(b * 128, 128)], sem_g)
              for b in range(8)]
        for g in gs:
            g.wait()
        ss = [pltpu.async_copy(vals.at[pl.ds(b * 128, 128)],
                               acc.at[dstb.at[b]], sem_sc, add=True)
              for b in range(8)]
        for s in ss:
            s.wait()
        return _

    lax.fori_loop(0, 25, _chunk, None)
    plsc.subcore_barrier()

    @pl.when(t < 15)
    def _():
        pltpu.sync_copy(acc.at[pl.ds(base, TPT)], zbuf.at[pl.ds(0, TPT)])
        pltpu.sync_copy(zbuf.at[pl.ds(0, TPT)],
                        p_hbm.at[pl.ds(c * N + base, TPT)])

    @pl.when(t == 15)
    def _():
        pltpu.sync_copy(acc.at[pl.ds(15 * TPT, LAST)], zbuf.at[pl.ds(0, LAST)])
        pltpu.sync_copy(zbuf.at[pl.ds(0, LAST)],
                        p_hbm.at[pl.ds(c * N + 15 * TPT, LAST)])


def _s1(x1, src_p, dst2d):
    return pl.kernel(
        _s1_body,
        out_type=jax.ShapeDtypeStruct((NC * N,), jnp.float32),
        mesh=_mesh,
        scratch_types=[
            pltpu.VMEM((1024,), jnp.int32),
            pltpu.VMEM((1024,), jnp.float32),
            pltpu.VMEM((8, 128), jnp.int32),
            pltpu.VMEM((TPT + 8,), jnp.float32),
            pltpu.VMEM_SHARED((ACC_PAD,), jnp.float32),
            pltpu.SemaphoreType.DMA,
            pltpu.SemaphoreType.DMA,
            pltpu.SemaphoreType.DMA,
        ],
    )(x1, src_p, dst2d)


QSZ = 12500
QPAD = 12544
RPT = 784
CH = 64


def _s2_body(u_hbm, src_hbm, dst_hbm, agg_hbm,
             srcb0, srcb1, dstb0, dstb1, offb0, offb1, rows0, rows1,
             acc, sem_st, sem_g, sem_sc):
    c = lax.axis_index("c")
    t = lax.axis_index("s")
    base = t * RPT
    lane = lax.broadcasted_iota(jnp.int32, (16,), 0)
    srcb = (srcb0, srcb1)
    dstb = (dstb0, dstb1)
    offb = (offb0, offb1)
    rows = (rows0, rows1)

    def _stage_fire(e0, lo, b):
        cp1 = pltpu.async_copy(src_hbm.at[pl.ds(e0, CH)], srcb[b], sem_st)
        cp2 = pltpu.async_copy(dst_hbm.at[pl.ds(e0, CH)], dstb[b], sem_st)
        cp1.wait()
        cp2.wait()
        for k in range(4):
            d = dstb[b][pl.ds(16 * k, 16)]
            off = d - lo
            inb = (off >= 0) & (off < QSZ)
            dummy = QSZ + lane + (16 * (k % 2))
            offb[b][pl.ds(16 * k, 16)] = jnp.where(inb, off, dummy)
        pltpu.async_copy(u_hbm.at[srcb[b]], rows[b], sem_g)

    def _wait_g(b):
        pltpu.make_async_copy(u_hbm.at[srcb[b]], rows[b], sem_g).wait()

    def _fire_sc(b):
        pltpu.async_copy(rows[b], acc.at[offb[b]], sem_sc, add=True)

    def _drain_sc(b):
        pltpu.make_async_copy(rows[b], acc.at[offb[b]], sem_sc).wait()

    for r in range(2):
        q = c + 2 * r
        lo = q * QSZ

        def _zr(i, _):
            for k in range(8):
                rows0[i, pl.ds(16 * k, 16)] = jnp.zeros((16,), jnp.float32)
            return _
        lax.fori_loop(0, CH, _zr, None)
        for k in range(12):
            pltpu.sync_copy(rows0, acc.at[pl.ds(base + k * CH, CH)])
        pltpu.sync_copy(rows0.at[pl.ds(0, 16)], acc.at[pl.ds(base + 768, 16)])
        plsc.subcore_barrier()

        ebase = t * 51200

        def _iter(j, _):
            e0 = ebase + j * (2 * CH)

            @pl.when(j > 0)
            def _():
                _drain_sc(0)
            _stage_fire(e0, lo, 0)

            @pl.when(j > 0)
            def _():
                _drain_sc(1)
            _stage_fire(e0 + CH, lo, 1)

            _wait_g(0)
            _fire_sc(0)
            _wait_g(1)
            _fire_sc(1)
            return _

        lax.fori_loop(0, 400, _iter, None)
        _drain_sc(0)
        _drain_sc(1)
        plsc.subcore_barrier()

        row_off = pl.multiple_of(q * QPAD + base, 8)
        for k in range(12):
            pltpu.sync_copy(acc.at[pl.ds(base + k * CH, CH)], rows0)
            pltpu.sync_copy(rows0, agg_hbm.at[pl.ds(row_off + k * CH, CH)])
        pltpu.sync_copy(acc.at[pl.ds(base + 768, 16)], rows0.at[pl.ds(0, 16)])
        pltpu.sync_copy(rows0.at[pl.ds(0, 16)],
                        agg_hbm.at[pl.ds(row_off + 768, 16)])


def _s2(u, src_p, dst_p):
    return pl.kernel(
        _s2_body,
        out_type=jax.ShapeDtypeStruct((4 * QPAD, 128), jnp.float32),
        mesh=_mesh,
        scratch_types=[
            pltpu.VMEM((CH,), jnp.int32),
            pltpu.VMEM((CH,), jnp.int32),
            pltpu.VMEM((CH,), jnp.int32),
            pltpu.VMEM((CH,), jnp.int32),
            pltpu.VMEM((CH,), jnp.int32),
            pltpu.VMEM((CH,), jnp.int32),
            pltpu.VMEM((CH, 128), jnp.float32),
            pltpu.VMEM((CH, 128), jnp.float32),
            pltpu.VMEM_SHARED((QPAD, 128), jnp.float32),
            pltpu.SemaphoreType.DMA,
            pltpu.SemaphoreType.DMA,
            pltpu.SemaphoreType.DMA,
        ],
    )(u, src_p, dst_p)


BN1 = 1000


def _t1_body(x_ref, p_ref, a1_ref, c1_ref, wf_ref, bf_ref, u_ref):
    s = x_ref[...] + p_ref[:, 0:1] + p_ref[:, 1:2]
    z1 = jnp.maximum(s * a1_ref[...] + c1_ref[...], 0.0)
    u_ref[...] = jnp.dot(z1, wf_ref[...],
                         preferred_element_type=jnp.float32) + bf_ref[...]


def _t1(x, p, A1, C1, Wf, bf):
    return pl.pallas_call(
        _t1_body,
        grid=(N // BN1,),
        in_specs=[
            pl.BlockSpec((BN1, 1), lambda i: (i, 0)),
            pl.BlockSpec((BN1, NC), lambda i: (i, 0)),
            pl.BlockSpec((1, 512), lambda i: (0, 0)),
            pl.BlockSpec((1, 512), lambda i: (0, 0)),
            pl.BlockSpec((512, 128), lambda i: (0, 0)),
            pl.BlockSpec((1, 128), lambda i: (0, 0)),
        ],
        out_specs=pl.BlockSpec((BN1, 128), lambda i: (i, 0)),
        out_shape=jax.ShapeDtypeStruct((N, 128), jnp.float32),
    )(x, p, A1, C1, Wf, bf)


def _t2_body(u_ref, a_ref, b2a_ref, g2_ref, be2_ref, wg_ref, cg_ref,
             wl2_ref, bl2_ref, out_ref):
    pre = u_ref[...] + a_ref[...] + b2a_ref[...]
    z2 = jnp.maximum(pre * g2_ref[...] + be2_ref[...], 0.0)
    tt = jnp.maximum(jnp.dot(z2, wg_ref[...],
                             preferred_element_type=jnp.float32)
                     + cg_ref[...], 0.0)
    o = jnp.dot(tt, wl2_ref[...],
                preferred_element_type=jnp.float32) + bl2_ref[...]
    m = jnp.max(o, axis=1, keepdims=True)
    lse = m + jnp.log(jnp.sum(jnp.exp(o - m), axis=1, keepdims=True))
    out_ref[...] = o - lse


def _t2(u, agg, b2a_r, G2, B2, Wg, cg, Wl2, bl2r):
    return pl.pallas_call(
        _t2_body,
        grid=(N // BN1,),
        in_specs=[
            pl.BlockSpec((BN1, 128), lambda i: (i, 0)),
            pl.BlockSpec((BN1, 128), lambda i: (i, 0)),
            pl.BlockSpec((1, 128), lambda i: (0, 0)),
            pl.BlockSpec((1, 128), lambda i: (0, 0)),
            pl.BlockSpec((1, 128), lambda i: (0, 0)),
            pl.BlockSpec((128, 16), lambda i: (0, 0)),
            pl.BlockSpec((1, 16), lambda i: (0, 0)),
            pl.BlockSpec((16, 6), lambda i: (0, 0)),
            pl.BlockSpec((1, 6), lambda i: (0, 0)),
        ],
        out_specs=pl.BlockSpec((BN1, 6), lambda i: (i, 0)),
        out_shape=jax.ShapeDtypeStruct((N, 6), jnp.float32),
    )(u, agg, b2a_r, G2, B2, Wg, cg, Wl2, bl2r)


def kernel(x, edge_attr, W1a, b1a, g1, be1, W1b, b1b, W2a, b2a, g2, be2,
           W2b, b2b, Wl1, bl1, Wl2, bl2, edge_index):
    cbn = np.float32(1.0 / np.sqrt(1.0 + 1e-5))

    A1 = (W1a[0] * g1 * cbn)[None, :]
    C1 = (b1a * g1 * cbn + be1)[None, :]
    Wf = W1b @ W2a
    bf = (b1b @ W2a)[None, :]
    b2a_r = b2a[None, :]
    G2 = (g2 * cbn)[None, :]
    B2 = be2[None, :]
    Wg = W2b @ Wl1
    cg = (b2b @ Wl1 + bl1)[None, :]
    bl2r = bl2[None, :]

    pad_i = jnp.arange(PAD, dtype=jnp.int32)
    src_p = jnp.concatenate([edge_index[0], pad_i % np.int32(N)])
    dst_p = jnp.concatenate([edge_index[1], N + (pad_i % np.int32(48))])
    dst2d = dst_p.reshape(NROWS, 128)
    x1 = x[:, 0]

    p = _s1(x1, src_p, dst2d)
    u = _t1(x, p.reshape(2, N).T, A1, C1, Wf, bf)
    aggp = _s2(u, src_p, dst_p)
    agg = aggp.reshape(4, QPAD, 128)[:, :QSZ].reshape(N, 128)
    out = _t2(u, agg, b2a_r, G2, B2, Wg, cg, Wl2, bl2r)
    return out
